# baseline (speedup 1.0000x reference)
import jax
import jax.numpy as jnp
from jax import lax
from jax.experimental import pallas as pl
from jax.experimental.pallas import tpu as pltpu

_BLOCK_M = 256
_NBUF = 4


def _adj_copy(adj_hbm, buf, sems, blk_idx, slot):
    return pltpu.make_async_copy(
        adj_hbm.at[pl.ds(blk_idx * _BLOCK_M, _BLOCK_M), :],
        buf.at[slot],
        sems.at[slot],
    )


def _gc_kernel(w_ref, b_ref, xt_ref, adj_hbm, outt_hbm,
               support_bf, outt, buf, sems, osem):
    n = adj_hbm.shape[0]
    nblk = n // _BLOCK_M
    for i in range(min(_NBUF, nblk)):
        _adj_copy(adj_hbm, buf, sems, i, i).start()
    support_bf[...] = (
        lax.dot_general(
            w_ref[...], xt_ref[...], (((1,), (0,)), ((), ())),
            preferred_element_type=jnp.float32,
        )
        + b_ref[...]
    ).astype(jnp.bfloat16)
    for i in range(nblk):
        slot = i % _NBUF
        _adj_copy(adj_hbm, buf, sems, i, slot).wait()
        adj_bf = buf[slot].astype(jnp.bfloat16)
        outt[:, pl.ds(i * _BLOCK_M, _BLOCK_M)] = lax.dot_general(
            support_bf[...], adj_bf, (((1,), (1,)), ((), ())),
            preferred_element_type=jnp.float32,
        )
        if i + _NBUF < nblk:
            _adj_copy(adj_hbm, buf, sems, i + _NBUF, slot).start()
    o_cp = pltpu.make_async_copy(outt, outt_hbm, osem)
    o_cp.start()
    o_cp.wait()


def kernel(input, adj, W, b):
    n, d_in = input.shape
    d_out = W.shape[0]
    outt = pl.pallas_call(
        _gc_kernel,
        in_specs=[
            pl.BlockSpec(memory_space=pltpu.MemorySpace.VMEM),
            pl.BlockSpec(memory_space=pltpu.MemorySpace.VMEM),
            pl.BlockSpec(memory_space=pltpu.MemorySpace.VMEM),
            pl.BlockSpec(memory_space=pltpu.MemorySpace.HBM),
        ],
        out_specs=pl.BlockSpec(memory_space=pltpu.MemorySpace.HBM),
        out_shape=jax.ShapeDtypeStruct((d_out, n), jnp.float32),
        scratch_shapes=[
            pltpu.VMEM((d_out, n), jnp.bfloat16),
            pltpu.VMEM((d_out, n), jnp.float32),
            pltpu.VMEM((_NBUF, _BLOCK_M, n), jnp.float32),
            pltpu.SemaphoreType.DMA((_NBUF,)),
            pltpu.SemaphoreType.DMA,
        ],
    )(W, b.reshape(d_out, 1), input.T, adj)
    return outt.T


# speedup vs baseline: 1.0192x; 1.0192x over previous
import jax
import jax.numpy as jnp
from jax import lax
from jax.experimental import pallas as pl
from jax.experimental.pallas import tpu as pltpu

_BLOCK_M = 512
_NBUF = 4


def _adj_copy(adj_hbm, buf, sems, blk_idx, slot):
    return pltpu.make_async_copy(
        adj_hbm.at[pl.ds(blk_idx * _BLOCK_M, _BLOCK_M), :],
        buf.at[slot],
        sems.at[slot],
    )


def _gc_kernel(w_ref, b_ref, xt_ref, adj_hbm, outt_hbm,
               support_bf, outt, buf, sems, osem):
    n = adj_hbm.shape[0]
    nblk = n // _BLOCK_M
    for i in range(min(_NBUF, nblk)):
        _adj_copy(adj_hbm, buf, sems, i, i).start()
    support_bf[...] = (
        lax.dot_general(
            w_ref[...], xt_ref[...], (((1,), (0,)), ((), ())),
            preferred_element_type=jnp.float32,
        )
        + b_ref[...]
    ).astype(jnp.bfloat16)
    for i in range(nblk):
        slot = i % _NBUF
        _adj_copy(adj_hbm, buf, sems, i, slot).wait()
        adj_bf = buf[slot].astype(jnp.bfloat16)
        outt[:, pl.ds(i * _BLOCK_M, _BLOCK_M)] = lax.dot_general(
            support_bf[...], adj_bf, (((1,), (1,)), ((), ())),
            preferred_element_type=jnp.float32,
        )
        if i + _NBUF < nblk:
            _adj_copy(adj_hbm, buf, sems, i + _NBUF, slot).start()
    o_cp = pltpu.make_async_copy(outt, outt_hbm, osem)
    o_cp.start()
    o_cp.wait()


def kernel(input, adj, W, b):
    n, d_in = input.shape
    d_out = W.shape[0]
    outt = pl.pallas_call(
        _gc_kernel,
        in_specs=[
            pl.BlockSpec(memory_space=pltpu.MemorySpace.VMEM),
            pl.BlockSpec(memory_space=pltpu.MemorySpace.VMEM),
            pl.BlockSpec(memory_space=pltpu.MemorySpace.VMEM),
            pl.BlockSpec(memory_space=pltpu.MemorySpace.HBM),
        ],
        out_specs=pl.BlockSpec(memory_space=pltpu.MemorySpace.HBM),
        out_shape=jax.ShapeDtypeStruct((d_out, n), jnp.float32),
        scratch_shapes=[
            pltpu.VMEM((d_out, n), jnp.bfloat16),
            pltpu.VMEM((d_out, n), jnp.float32),
            pltpu.VMEM((_NBUF, _BLOCK_M, n), jnp.float32),
            pltpu.SemaphoreType.DMA((_NBUF,)),
            pltpu.SemaphoreType.DMA,
        ],
    )(W, b.reshape(d_out, 1), input.T, adj)
    return outt.T
